# baseline (speedup 1.0000x reference)
import functools

import jax
import jax.numpy as jnp
import numpy as np
from jax import lax
from jax.experimental import pallas as pl
from jax.experimental.pallas import tpu as pltpu
from jax.experimental.pallas import tpu_sc as plsc

RADIUS = 0.3
NSAMPLE = 32
B, N, NP, C = 4, 16384, 512, 128
RSQ = np.float32(RADIUS ** 2)

L = 16
NW = 32
Q = B * NP
QPT = Q // NW
NCHUNK = N // L
QBUF = 384
UNROLL = 8
NROWS = C + 3
RPT = 17


def _ball_query_body(xh, yh, zh, bph, poseh, maskh, qxh, qyh, qzh, aqh,
                     idx_out, poseg_out, maskg_out,
                     x_v, y_v, z_v, bp_v, pose_v, mask_v,
                     qx_v, qy_v, qz_v, aq_v, qidx_v,
                     s_idx, s_pose, s_mask):
    cid = lax.axis_index("c")
    sid = lax.axis_index("s")
    wid = sid * 2 + cid
    b = wid // (NW // B)
    q0 = (wid % (NW // B)) * QPT

    pltpu.sync_copy(xh.at[b], x_v)
    pltpu.sync_copy(yh.at[b], y_v)
    pltpu.sync_copy(zh.at[b], z_v)
    pltpu.sync_copy(bph.at[b], bp_v)
    pltpu.sync_copy(poseh.at[b], pose_v)
    pltpu.sync_copy(maskh.at[b], mask_v)
    pltpu.sync_copy(qxh.at[b, pl.ds(q0, QPT)], qx_v.at[pl.ds(0, QPT)])
    pltpu.sync_copy(qyh.at[b, pl.ds(q0, QPT)], qy_v.at[pl.ds(0, QPT)])
    pltpu.sync_copy(qzh.at[b, pl.ds(q0, QPT)], qz_v.at[pl.ds(0, QPT)])
    pltpu.sync_copy(aqh.at[b, pl.ds(q0, QPT)], aq_v.at[pl.ds(0, QPT)])

    lane = lax.iota(jnp.int32, L)
    zeros16 = jnp.zeros((L,), jnp.int32)

    def q_body(q, carry):
        qsp = jnp.full((L,), q, jnp.int32)
        qx = plsc.load_gather(qx_v, [qsp])
        qy = plsc.load_gather(qy_v, [qsp])
        qz = plsc.load_gather(qz_v, [qsp])
        aq = plsc.load_gather(aq_v, [qsp])

        def cond(c):
            j, cnt = c
            return (j < NCHUNK // UNROLL) & jnp.all(cnt < NSAMPLE)

        def body(c):
            j, cnt = c
            masks = []
            pidxs = []
            for u in range(UNROLL):
                base = (j * UNROLL + u) * L
                px = x_v[pl.ds(base, L)]
                py = y_v[pl.ds(base, L)]
                pz = z_v[pl.ds(base, L)]
                bp = bp_v[pl.ds(base, L)]
                p0 = qx * px
                p1 = qy * py
                p2 = qz * pz
                s1 = p0 + p1
                t1 = s1 - p0
                e1 = (p0 - (s1 - t1)) + (p1 - t1)
                s2 = s1 + p2
                t2 = s2 - s1
                e2 = (s1 - (s2 - t2)) + (p2 - t2)
                s3 = s2 + (e1 + e2)
                d2 = (aq + bp) - 2.0 * s3
                masks.append(d2 < RSQ)
                pidxs.append(base + lane)
            pcs = [plsc.all_reduce_population_count(m) for m in masks]
            off = cnt
            for u in range(UNROLL):
                cs = plsc.cumsum(masks[u].astype(jnp.int32))
                pos = off + cs - 1
                plsc.store_scatter(qidx_v, [pos], pidxs[u], mask=masks[u])
                off = off + pcs[u]
            return j + 1, off

        cnt0 = jnp.zeros((L,), jnp.int32)
        _, cnt = lax.while_loop(cond, body, (jnp.int32(0), cnt0))

        iv0 = qidx_v[pl.ds(0, L)]
        iv1 = qidx_v[pl.ds(L, L)]
        first = plsc.load_gather(qidx_v, [zeros16])
        first = jnp.where(cnt > 0, first, 0)
        iv0 = jnp.where(lane < cnt, iv0, first)
        iv1 = jnp.where((lane + L) < cnt, iv1, first)

        o = q * NSAMPLE
        s_idx[pl.ds(o, L)] = iv0
        s_idx[pl.ds(o + L, L)] = iv1
        s_pose[pl.ds(o, L)] = plsc.load_gather(pose_v, [iv0])
        s_pose[pl.ds(o + L, L)] = plsc.load_gather(pose_v, [iv1])
        s_mask[pl.ds(o, L)] = plsc.load_gather(mask_v, [iv0])
        s_mask[pl.ds(o + L, L)] = plsc.load_gather(mask_v, [iv1])
        return carry

    lax.fori_loop(0, QPT, q_body, 0)

    g0 = wid * (QPT * NSAMPLE)
    pltpu.sync_copy(s_idx, idx_out.at[pl.ds(g0, QPT * NSAMPLE)])
    pltpu.sync_copy(s_pose, poseg_out.at[pl.ds(g0, QPT * NSAMPLE)])
    pltpu.sync_copy(s_mask, maskg_out.at[pl.ds(g0, QPT * NSAMPLE)])


def _gather_body(feat_h, aux_h, idxf_h, gf_h, gaux_h, idx_v,
                 row0, row1, og0, og1, isem0, isem1, osem0, osem1):
    cid = lax.axis_index("c")
    sid = lax.axis_index("s")
    wid = sid * 2 + cid
    b = wid // (NW // B)
    slot = wid % (NW // B)
    c0 = slot * RPT
    ntask = jnp.minimum(RPT, NROWS - c0)

    rows = (row0, row1)
    ogs = (og0, og1)
    isems = (isem0, isem1)
    osems = (osem0, osem1)

    pltpu.sync_copy(idxf_h.at[pl.ds(b * NP * NSAMPLE, NP * NSAMPLE)], idx_v)

    def start_in(t, buf, sem):
        ch = c0 + t

        @pl.when(ch < C)
        def _():
            pltpu.async_copy(feat_h.at[b, ch], buf, sem)

        @pl.when(ch >= C)
        def _():
            pltpu.async_copy(aux_h.at[b, jnp.maximum(ch - C, 0)], buf, sem)

    def start_out(t, buf, sem):
        ch = c0 + t

        @pl.when(ch < C)
        def _():
            pltpu.async_copy(buf, gf_h.at[b, ch + 3], sem)

        @pl.when(ch >= C)
        def _():
            pltpu.async_copy(buf, gaux_h.at[b, jnp.maximum(ch - C, 0)], sem)

    def wait_in(buf, sem):
        pltpu.make_async_copy(feat_h.at[b, 0], buf, sem).wait()

    def wait_out(buf, sem):
        pltpu.make_async_copy(buf, gf_h.at[b, 0], sem).wait()

    @pl.when(0 < ntask)
    def _():
        start_in(0, rows[0], isems[0])

    for t in range(RPT):
        p = t % 2

        @pl.when(t < ntask)
        def _(t=t, p=p):
            wait_in(rows[p], isems[p])
            if t + 1 < RPT:
                @pl.when(t + 1 < ntask)
                def _():
                    start_in(t + 1, rows[1 - p], isems[1 - p])
            if t >= 2:
                wait_out(ogs[p], osems[p])

            def g(i, c2):
                iv = idx_v[pl.ds(i * L, L)]
                ogs[p][pl.ds(i * L, L)] = plsc.load_gather(rows[p], [iv])
                return c2

            lax.fori_loop(0, NP * NSAMPLE // L, g, 0, unroll=8)
            start_out(t, ogs[p], osems[p])

    for t in range(RPT):
        p = t % 2

        @pl.when((t < ntask) & (t >= ntask - 2))
        def _(t=t, p=p):
            wait_out(ogs[p], osems[p])


def _mode_rot_body(nf_ref, idx_ref, poseg_ref, maskg_ref, gx_ref, gy_ref,
                   gz_ref, nxyz_ref, rb_ref, rf_ref, br_ref, mp_ref):
    del nf_ref
    idx = idx_ref[...]
    lane = lax.broadcasted_iota(jnp.int32, (Q, NSAMPLE), 1)
    filler = (idx == idx[:, 0:1]) & (lane > 0)
    m = maskg_ref[...] != 0.0
    cp = jnp.where(m, poseg_ref[...], 7.0 + rb_ref[...])
    cp = jnp.where(filler, 11.0 + rf_ref[...], cp)
    valid = m & jnp.logical_not(filler)

    cnt = jnp.zeros((Q, NSAMPLE), jnp.int32)
    for j in range(NSAMPLE):
        cnt = cnt + (cp == cp[:, j:j + 1]).astype(jnp.int32)
    maxc = jnp.max(cnt, axis=1, keepdims=True)
    cand = jnp.where(cnt == maxc, cp, jnp.inf)
    mode = jnp.min(cand, axis=1, keepdims=True)
    eqm = cp == mode
    pos = jnp.min(jnp.where(eqm, lane, NSAMPLE + 1), axis=1, keepdims=True)
    isfg = jnp.max((valid & (lane == pos)).astype(jnp.int32),
                   axis=1, keepdims=True)
    modep = jnp.where(isfg > 0, mode, 0.0)
    mp_ref[...] = modep

    ang = -modep
    cth = jnp.cos(ang)
    sth = jnp.sin(ang)
    qx = nxyz_ref[:, 0:1]
    qy = nxyz_ref[:, 1:2]
    qz = nxyz_ref[:, 2:3]
    xf = (gx_ref[...] - qx) / RADIUS
    yf = (gy_ref[...] - qy) / RADIUS
    zf = (gz_ref[...] - qz) / RADIUS
    ox = cth * xf + (-sth) * yf
    oy = sth * xf + cth * yf
    for bb in range(B):
        r = slice(bb * NP, (bb + 1) * NP)
        br_ref[bb, 0] = ox[r, :]
        br_ref[bb, 1] = oy[r, :]
        br_ref[bb, 2] = zf[r, :]


def _round_bf16(x):
    u = lax.bitcast_convert_type(x, jnp.uint32)
    r = (u + 0x7FFF + ((u >> 16) & 1)) & np.uint32(0xFFFF0000)
    return lax.bitcast_convert_type(r, jnp.float32)


def kernel(xyz, new_xyz, features, point_pose, point_pose_mask):
    xt = jnp.transpose(xyz, (0, 2, 1))
    xh, yh, zh = xt[:, 0], xt[:, 1], xt[:, 2]
    xb, yb, zb = _round_bf16(xh), _round_bf16(yh), _round_bf16(zh)
    bph = jnp.sum(xyz ** 2, axis=-1)
    nt = jnp.transpose(new_xyz, (0, 2, 1))
    qxb, qyb, qzb = (_round_bf16(nt[:, 0]), _round_bf16(nt[:, 1]),
                     _round_bf16(nt[:, 2]))
    aqh = jnp.sum(new_xyz ** 2, axis=-1)
    maskf = point_pose_mask.astype(jnp.float32)
    aux = jnp.stack([xh, yh, zh], axis=1)

    key = jax.random.key(42)
    kb, kf = jax.random.split(key)
    rb = jax.random.uniform(kb, (B, NP, NSAMPLE), dtype=jnp.float32)
    rf = jax.random.uniform(kf, (B, NP, NSAMPLE), dtype=jnp.float32)

    mesh = plsc.VectorSubcoreMesh(core_axis_name="c", subcore_axis_name="s")
    sc_params = pltpu.CompilerParams(needs_layout_passes=False)
    flat = jax.ShapeDtypeStruct((Q * NSAMPLE,), jnp.float32)
    flat_i = jax.ShapeDtypeStruct((Q * NSAMPLE,), jnp.int32)

    k1 = pl.kernel(
        _ball_query_body,
        out_type=[flat_i, flat, flat],
        mesh=mesh,
        compiler_params=sc_params,
        scratch_types=[
            pltpu.VMEM((N,), jnp.float32),
            pltpu.VMEM((N,), jnp.float32),
            pltpu.VMEM((N,), jnp.float32),
            pltpu.VMEM((N,), jnp.float32),
            pltpu.VMEM((N,), jnp.float32),
            pltpu.VMEM((N,), jnp.float32),
            pltpu.VMEM((128,), jnp.float32),
            pltpu.VMEM((128,), jnp.float32),
            pltpu.VMEM((128,), jnp.float32),
            pltpu.VMEM((128,), jnp.float32),
            pltpu.VMEM((QBUF,), jnp.int32),
            pltpu.VMEM((QPT * NSAMPLE,), jnp.int32),
            pltpu.VMEM((QPT * NSAMPLE,), jnp.float32),
            pltpu.VMEM((QPT * NSAMPLE,), jnp.float32),
        ],
    )
    idx_f, poseg, maskg = k1(xb, yb, zb, bph, point_pose, maskf,
                             qxb, qyb, qzb, aqh)

    k2 = pl.kernel(
        _gather_body,
        out_type=[
            jax.ShapeDtypeStruct((B, C + 3, NP * NSAMPLE), jnp.float32),
            jax.ShapeDtypeStruct((B, 3, NP * NSAMPLE), jnp.float32),
        ],
        mesh=mesh,
        compiler_params=sc_params,
        scratch_types=[
            pltpu.VMEM((NP * NSAMPLE,), jnp.int32),
            pltpu.VMEM((N,), jnp.float32),
            pltpu.VMEM((N,), jnp.float32),
            pltpu.VMEM((NP * NSAMPLE,), jnp.float32),
            pltpu.VMEM((NP * NSAMPLE,), jnp.float32),
            pltpu.SemaphoreType.DMA,
            pltpu.SemaphoreType.DMA,
            pltpu.SemaphoreType.DMA,
            pltpu.SemaphoreType.DMA,
        ],
    )
    gf, gaux = k2(features, aux, idx_f)

    NPK = NP * NSAMPLE
    full = lambda shape: pl.BlockSpec(shape, lambda i: (0,) * len(shape))
    nf_out, mp = pl.pallas_call(
        _mode_rot_body,
        grid=(1,),
        in_specs=[
            pl.BlockSpec(memory_space=pl.ANY),
            full((Q, NSAMPLE)), full((Q, NSAMPLE)), full((Q, NSAMPLE)),
            full((Q, NSAMPLE)), full((Q, NSAMPLE)), full((Q, NSAMPLE)),
            full((Q, 3)), full((Q, NSAMPLE)), full((Q, NSAMPLE)),
        ],
        out_specs=[
            pl.BlockSpec((B, 3, NP, NSAMPLE), lambda i: (0, 0, 0, 0)),
            full((Q, 1)),
        ],
        out_shape=[
            jax.ShapeDtypeStruct((B, C + 3, NP, NSAMPLE), jnp.float32),
            jax.ShapeDtypeStruct((Q, 1), jnp.float32),
        ],
        input_output_aliases={0: 0},
    )(gf.reshape(B, C + 3, NP, NSAMPLE),
      idx_f.reshape(Q, NSAMPLE),
      poseg.reshape(Q, NSAMPLE),
      maskg.reshape(Q, NSAMPLE),
      gaux[:, 0].reshape(Q, NSAMPLE),
      gaux[:, 1].reshape(Q, NSAMPLE),
      gaux[:, 2].reshape(Q, NSAMPLE),
      new_xyz.reshape(Q, 3),
      rb.reshape(Q, NSAMPLE),
      rf.reshape(Q, NSAMPLE))

    return nf_out, mp.reshape(B, NP)


# speedup vs baseline: 1.2914x; 1.2914x over previous
import functools

import jax
import jax.numpy as jnp
import numpy as np
from jax import lax
from jax.experimental import pallas as pl
from jax.experimental.pallas import tpu as pltpu
from jax.experimental.pallas import tpu_sc as plsc

RADIUS = 0.3
NSAMPLE = 32
B, N, NP, C = 4, 16384, 512, 128
RSQ = np.float32(RADIUS ** 2)

L = 16
NW = 32
Q = B * NP
QPT = Q // NW
NCHUNK = N // L
QBUF = 384
UNROLL = 8
NROWS = C + 3
RPT = 17


def _ball_query_body(xh, yh, zh, bph, poseh, maskh, qxh, qyh, qzh, aqh,
                     idx_out, poseg_out, maskg_out,
                     x_v, y_v, z_v, bp_v, pose_v, mask_v,
                     qx_v, qy_v, qz_v, aq_v, qidx_v,
                     s_idx, s_pose, s_mask):
    cid = lax.axis_index("c")
    sid = lax.axis_index("s")
    wid = sid * 2 + cid
    b = wid // (NW // B)
    q0 = (wid % (NW // B)) * QPT

    pltpu.sync_copy(xh.at[b], x_v)
    pltpu.sync_copy(yh.at[b], y_v)
    pltpu.sync_copy(zh.at[b], z_v)
    pltpu.sync_copy(bph.at[b], bp_v)
    pltpu.sync_copy(poseh.at[b], pose_v)
    pltpu.sync_copy(maskh.at[b], mask_v)
    pltpu.sync_copy(qxh.at[b, pl.ds(q0, QPT)], qx_v.at[pl.ds(0, QPT)])
    pltpu.sync_copy(qyh.at[b, pl.ds(q0, QPT)], qy_v.at[pl.ds(0, QPT)])
    pltpu.sync_copy(qzh.at[b, pl.ds(q0, QPT)], qz_v.at[pl.ds(0, QPT)])
    pltpu.sync_copy(aqh.at[b, pl.ds(q0, QPT)], aq_v.at[pl.ds(0, QPT)])

    lane = lax.iota(jnp.int32, L)
    zeros16 = jnp.zeros((L,), jnp.int32)

    def q_body(q, carry):
        qsp = jnp.full((L,), q, jnp.int32)
        qx = plsc.load_gather(qx_v, [qsp])
        qy = plsc.load_gather(qy_v, [qsp])
        qz = plsc.load_gather(qz_v, [qsp])
        aq = plsc.load_gather(aq_v, [qsp])

        def cond(c):
            j, cnt = c
            return (j < NCHUNK // UNROLL) & jnp.all(cnt < NSAMPLE)

        def body(c):
            j, cnt = c
            masks = []
            pidxs = []
            for u in range(UNROLL):
                base = (j * UNROLL + u) * L
                px = x_v[pl.ds(base, L)]
                py = y_v[pl.ds(base, L)]
                pz = z_v[pl.ds(base, L)]
                bp = bp_v[pl.ds(base, L)]
                p0 = qx * px
                p1 = qy * py
                p2 = qz * pz
                s1 = p0 + p1
                t1 = s1 - p0
                e1 = (p0 - (s1 - t1)) + (p1 - t1)
                s2 = s1 + p2
                t2 = s2 - s1
                e2 = (s1 - (s2 - t2)) + (p2 - t2)
                s3 = s2 + (e1 + e2)
                d2 = (aq + bp) - 2.0 * s3
                masks.append(d2 < RSQ)
                pidxs.append(base + lane)
            pcs = [plsc.all_reduce_population_count(m) for m in masks]
            off = cnt
            for u in range(UNROLL):
                cs = plsc.cumsum(masks[u].astype(jnp.int32))
                pos = off + cs - 1
                plsc.store_scatter(qidx_v, [pos], pidxs[u], mask=masks[u])
                off = off + pcs[u]
            return j + 1, off

        cnt0 = jnp.zeros((L,), jnp.int32)
        _, cnt = lax.while_loop(cond, body, (jnp.int32(0), cnt0))

        iv0 = qidx_v[pl.ds(0, L)]
        iv1 = qidx_v[pl.ds(L, L)]
        first = plsc.load_gather(qidx_v, [zeros16])
        first = jnp.where(cnt > 0, first, 0)
        iv0 = jnp.where(lane < cnt, iv0, first)
        iv1 = jnp.where((lane + L) < cnt, iv1, first)

        o = q * NSAMPLE
        s_idx[pl.ds(o, L)] = iv0
        s_idx[pl.ds(o + L, L)] = iv1
        s_pose[pl.ds(o, L)] = plsc.load_gather(pose_v, [iv0])
        s_pose[pl.ds(o + L, L)] = plsc.load_gather(pose_v, [iv1])
        s_mask[pl.ds(o, L)] = plsc.load_gather(mask_v, [iv0])
        s_mask[pl.ds(o + L, L)] = plsc.load_gather(mask_v, [iv1])
        return carry

    lax.fori_loop(0, QPT, q_body, 0)

    g0 = wid * (QPT * NSAMPLE)
    pltpu.sync_copy(s_idx, idx_out.at[pl.ds(g0, QPT * NSAMPLE)])
    pltpu.sync_copy(s_pose, poseg_out.at[pl.ds(g0, QPT * NSAMPLE)])
    pltpu.sync_copy(s_mask, maskg_out.at[pl.ds(g0, QPT * NSAMPLE)])


def _gather_body(feat_h, aux_h, idxf_h, gf_h, gaux_h, idx_v,
                 row0, row1, og0, og1, isem0, isem1, osem0, osem1):
    cid = lax.axis_index("c")
    sid = lax.axis_index("s")
    wid = sid * 2 + cid
    b = wid // (NW // B)
    slot = wid % (NW // B)
    c0 = slot * RPT
    ntask = jnp.minimum(RPT, NROWS - c0)

    rows = (row0, row1)
    ogs = (og0, og1)
    isems = (isem0, isem1)
    osems = (osem0, osem1)

    pltpu.sync_copy(idxf_h.at[pl.ds(b * NP * NSAMPLE, NP * NSAMPLE)], idx_v)

    def start_in(t, buf, sem):
        ch = c0 + t

        @pl.when(ch < C)
        def _():
            pltpu.async_copy(feat_h.at[b, ch], buf, sem)

        @pl.when(ch >= C)
        def _():
            pltpu.async_copy(aux_h.at[b, jnp.maximum(ch - C, 0)], buf, sem)

    def start_out(t, buf, sem):
        ch = c0 + t

        @pl.when(ch < C)
        def _():
            pltpu.async_copy(buf, gf_h.at[b, ch], sem)

        @pl.when(ch >= C)
        def _():
            pltpu.async_copy(buf, gaux_h.at[b, jnp.maximum(ch - C, 0)], sem)

    def wait_in(buf, sem):
        pltpu.make_async_copy(feat_h.at[b, 0], buf, sem).wait()

    def wait_out(buf, sem):
        pltpu.make_async_copy(buf, gf_h.at[b, 0], sem).wait()

    @pl.when(0 < ntask)
    def _():
        start_in(0, rows[0], isems[0])

    for t in range(RPT):
        p = t % 2

        @pl.when(t < ntask)
        def _(t=t, p=p):
            wait_in(rows[p], isems[p])
            if t + 1 < RPT:
                @pl.when(t + 1 < ntask)
                def _():
                    start_in(t + 1, rows[1 - p], isems[1 - p])
            if t >= 2:
                wait_out(ogs[p], osems[p])

            def g(i, c2):
                iv = idx_v[pl.ds(i * L, L)]
                ogs[p][pl.ds(i * L, L)] = plsc.load_gather(rows[p], [iv])
                return c2

            lax.fori_loop(0, NP * NSAMPLE // L, g, 0, unroll=8)
            start_out(t, ogs[p], osems[p])

    for t in range(RPT):
        p = t % 2

        @pl.when((t < ntask) & (t >= ntask - 2))
        def _(t=t, p=p):
            wait_out(ogs[p], osems[p])


def _mode_rot_body(idx_ref, poseg_ref, maskg_ref, gx_ref, gy_ref,
                   gz_ref, nxyz_ref, rb_ref, rf_ref, br_ref, mp_ref):
    idx = idx_ref[...]
    lane = lax.broadcasted_iota(jnp.int32, (Q, NSAMPLE), 1)
    filler = (idx == idx[:, 0:1]) & (lane > 0)
    m = maskg_ref[...] != 0.0
    cp = jnp.where(m, poseg_ref[...], 7.0 + rb_ref[...])
    cp = jnp.where(filler, 11.0 + rf_ref[...], cp)
    valid = m & jnp.logical_not(filler)

    cnt = jnp.zeros((Q, NSAMPLE), jnp.int32)
    for j in range(NSAMPLE):
        cnt = cnt + (cp == cp[:, j:j + 1]).astype(jnp.int32)
    maxc = jnp.max(cnt, axis=1, keepdims=True)
    cand = jnp.where(cnt == maxc, cp, jnp.inf)
    mode = jnp.min(cand, axis=1, keepdims=True)
    eqm = cp == mode
    pos = jnp.min(jnp.where(eqm, lane, NSAMPLE + 1), axis=1, keepdims=True)
    isfg = jnp.max((valid & (lane == pos)).astype(jnp.int32),
                   axis=1, keepdims=True)
    modep = jnp.where(isfg > 0, mode, 0.0)
    mp_ref[...] = modep

    ang = -modep
    cth = jnp.cos(ang)
    sth = jnp.sin(ang)
    qx = nxyz_ref[:, 0:1]
    qy = nxyz_ref[:, 1:2]
    qz = nxyz_ref[:, 2:3]
    xf = (gx_ref[...] - qx) / RADIUS
    yf = (gy_ref[...] - qy) / RADIUS
    zf = (gz_ref[...] - qz) / RADIUS
    ox = cth * xf + (-sth) * yf
    oy = sth * xf + cth * yf
    for bb in range(B):
        r = slice(bb * NP, (bb + 1) * NP)
        br_ref[bb, 0] = ox[r, :]
        br_ref[bb, 1] = oy[r, :]
        br_ref[bb, 2] = zf[r, :]


def _round_bf16(x):
    u = lax.bitcast_convert_type(x, jnp.uint32)
    r = (u + 0x7FFF + ((u >> 16) & 1)) & np.uint32(0xFFFF0000)
    return lax.bitcast_convert_type(r, jnp.float32)


def kernel(xyz, new_xyz, features, point_pose, point_pose_mask):
    xt = jnp.transpose(xyz, (0, 2, 1))
    xh, yh, zh = xt[:, 0], xt[:, 1], xt[:, 2]
    xb, yb, zb = _round_bf16(xh), _round_bf16(yh), _round_bf16(zh)
    bph = jnp.sum(xyz ** 2, axis=-1)
    nt = jnp.transpose(new_xyz, (0, 2, 1))
    qxb, qyb, qzb = (_round_bf16(nt[:, 0]), _round_bf16(nt[:, 1]),
                     _round_bf16(nt[:, 2]))
    aqh = jnp.sum(new_xyz ** 2, axis=-1)
    maskf = point_pose_mask.astype(jnp.float32)
    aux = jnp.stack([xh, yh, zh], axis=1)

    key = jax.random.key(42)
    kb, kf = jax.random.split(key)
    rb = jax.random.uniform(kb, (B, NP, NSAMPLE), dtype=jnp.float32)
    rf = jax.random.uniform(kf, (B, NP, NSAMPLE), dtype=jnp.float32)

    mesh = plsc.VectorSubcoreMesh(core_axis_name="c", subcore_axis_name="s")
    sc_params = pltpu.CompilerParams(needs_layout_passes=False)
    flat = jax.ShapeDtypeStruct((Q * NSAMPLE,), jnp.float32)
    flat_i = jax.ShapeDtypeStruct((Q * NSAMPLE,), jnp.int32)

    k1 = pl.kernel(
        _ball_query_body,
        out_type=[flat_i, flat, flat],
        mesh=mesh,
        compiler_params=sc_params,
        scratch_types=[
            pltpu.VMEM((N,), jnp.float32),
            pltpu.VMEM((N,), jnp.float32),
            pltpu.VMEM((N,), jnp.float32),
            pltpu.VMEM((N,), jnp.float32),
            pltpu.VMEM((N,), jnp.float32),
            pltpu.VMEM((N,), jnp.float32),
            pltpu.VMEM((128,), jnp.float32),
            pltpu.VMEM((128,), jnp.float32),
            pltpu.VMEM((128,), jnp.float32),
            pltpu.VMEM((128,), jnp.float32),
            pltpu.VMEM((QBUF,), jnp.int32),
            pltpu.VMEM((QPT * NSAMPLE,), jnp.int32),
            pltpu.VMEM((QPT * NSAMPLE,), jnp.float32),
            pltpu.VMEM((QPT * NSAMPLE,), jnp.float32),
        ],
    )
    idx_f, poseg, maskg = k1(xb, yb, zb, bph, point_pose, maskf,
                             qxb, qyb, qzb, aqh)

    k2 = pl.kernel(
        _gather_body,
        out_type=[
            jax.ShapeDtypeStruct((B, C, NP * NSAMPLE), jnp.float32),
            jax.ShapeDtypeStruct((B, 3, NP * NSAMPLE), jnp.float32),
        ],
        mesh=mesh,
        compiler_params=sc_params,
        scratch_types=[
            pltpu.VMEM((NP * NSAMPLE,), jnp.int32),
            pltpu.VMEM((N,), jnp.float32),
            pltpu.VMEM((N,), jnp.float32),
            pltpu.VMEM((NP * NSAMPLE,), jnp.float32),
            pltpu.VMEM((NP * NSAMPLE,), jnp.float32),
            pltpu.SemaphoreType.DMA,
            pltpu.SemaphoreType.DMA,
            pltpu.SemaphoreType.DMA,
            pltpu.SemaphoreType.DMA,
        ],
    )
    gf, gaux = k2(features, aux, idx_f)

    br, mp = pl.pallas_call(
        _mode_rot_body,
        out_shape=[
            jax.ShapeDtypeStruct((B, 3, NP, NSAMPLE), jnp.float32),
            jax.ShapeDtypeStruct((Q, 1), jnp.float32),
        ],
    )(idx_f.reshape(Q, NSAMPLE),
      poseg.reshape(Q, NSAMPLE),
      maskg.reshape(Q, NSAMPLE),
      gaux[:, 0].reshape(Q, NSAMPLE),
      gaux[:, 1].reshape(Q, NSAMPLE),
      gaux[:, 2].reshape(Q, NSAMPLE),
      new_xyz.reshape(Q, 3),
      rb.reshape(Q, NSAMPLE),
      rf.reshape(Q, NSAMPLE))

    new_features = jnp.concatenate(
        [br, gf.reshape(B, C, NP, NSAMPLE)], axis=1)
    return new_features, mp.reshape(B, NP)
